# Initial kernel scaffold; baseline (speedup 1.0000x reference)
import jax
import jax.numpy as jnp
from jax.experimental import pallas as pl


def kernel(x, emb, W, b):
    raise NotImplementedError("write your pallas kernel here")


# speedup vs baseline: 1.1827x; 1.1827x over previous
import functools

import jax
import jax.numpy as jnp
from jax import lax
from jax.experimental import pallas as pl
from jax.experimental.pallas import tpu as pltpu
from jax.experimental.pallas import tpu_sc as plsc

_VOCAB = 1000000
_EMBED = 64
_B = 16384
_L = 50
_N = _B * _L

_NC = 2
_NS = 16
_NW = _NC * _NS
_KC = 128
_PER_W = _N // _NW
_NCHUNK = _PER_W // _KC


def _gather_body(emb_hbm, idx_hbm, out_hbm, idx_v, rows0, rows1, sem0, sem1):
    wid = lax.axis_index("s") * _NC + lax.axis_index("c")
    base = wid * _PER_W
    pltpu.sync_copy(idx_hbm.at[wid], idx_v)

    pltpu.make_async_copy(emb_hbm.at[idx_v.at[0]], rows0, sem0).start()
    pltpu.make_async_copy(emb_hbm.at[idx_v.at[1]], rows1, sem1).start()

    def body(g, carry):
        j0 = 2 * g
        pltpu.make_async_copy(emb_hbm.at[idx_v.at[j0]], rows0, sem0).wait()
        pltpu.sync_copy(rows0, out_hbm.at[pl.ds(base + j0 * _KC, _KC)])

        @pl.when(j0 + 2 < _NCHUNK)
        def _():
            pltpu.make_async_copy(emb_hbm.at[idx_v.at[j0 + 2]], rows0, sem0).start()

        j1 = j0 + 1
        pltpu.make_async_copy(emb_hbm.at[idx_v.at[j1]], rows1, sem1).wait()
        pltpu.sync_copy(rows1, out_hbm.at[pl.ds(base + j1 * _KC, _KC)])

        @pl.when(j1 + 2 < _NCHUNK)
        def _():
            pltpu.make_async_copy(emb_hbm.at[idx_v.at[j1 + 2]], rows1, sem1).start()

        return carry

    lax.fori_loop(0, _NCHUNK // 2, body, 0)


_gather = functools.partial(
    pl.kernel,
    out_type=jax.ShapeDtypeStruct((_N, _EMBED), jnp.float32),
    mesh=plsc.VectorSubcoreMesh(core_axis_name="c", subcore_axis_name="s"),
    scratch_types=[
        pltpu.VMEM((_NCHUNK, _KC), jnp.int32),
        pltpu.VMEM((_KC, _EMBED), jnp.float32),
        pltpu.VMEM((_KC, _EMBED), jnp.float32),
        pltpu.SemaphoreType.DMA,
        pltpu.SemaphoreType.DMA,
    ],
    compiler_params=pltpu.CompilerParams(use_tc_tiling_on_sc=False),
)(_gather_body)


_BLK = 8192


def _linear_body(h_ref, wt_ref, b_ref, o_ref):
    o_ref[...] = (
        jnp.dot(h_ref[...], wt_ref[...], preferred_element_type=jnp.float32)
        + b_ref[...]
    )


def _linear(h, wt, b2):
    return pl.pallas_call(
        _linear_body,
        grid=(_N // _BLK,),
        in_specs=[
            pl.BlockSpec((_BLK, _EMBED), lambda i: (i, 0)),
            pl.BlockSpec((_EMBED, _EMBED), lambda i: (0, 0)),
            pl.BlockSpec((1, _EMBED), lambda i: (0, 0)),
        ],
        out_specs=pl.BlockSpec((_BLK, _EMBED), lambda i: (i, 0)),
        out_shape=jax.ShapeDtypeStruct((_N, _EMBED), jnp.float32),
    )(h, wt, b2)


def kernel(x, emb, W, b):
    idx = x.reshape(-1).astype(jnp.int32).reshape(_NW, _NCHUNK, _KC)
    h = _gather(emb, idx)
    out = _linear(h, W.T, b.reshape(1, _EMBED))
    return out.reshape(_B, _L, _EMBED)


# speedup vs baseline: 1.5775x; 1.3338x over previous
import functools

import jax
import jax.numpy as jnp
from jax import lax
from jax.experimental import pallas as pl
from jax.experimental.pallas import tpu as pltpu
from jax.experimental.pallas import tpu_sc as plsc

_VOCAB = 1000000
_EMBED = 64
_B = 16384
_L = 50
_N = _B * _L

_NC = 2
_NS = 16
_NW = _NC * _NS
_KC = 128
_PER_W = _N // _NW
_NCHUNK = _PER_W // _KC


def _gather_body(emb_hbm, idx_hbm, out_hbm, idx_v, rows0, rows1,
                 gs0, gs1, ws0, ws1):
    wid = lax.axis_index("s") * _NC + lax.axis_index("c")
    base = wid * _PER_W
    pltpu.sync_copy(idx_hbm.at[wid], idx_v)

    pltpu.make_async_copy(emb_hbm.at[idx_v.at[0]], rows0, gs0).start()
    pltpu.make_async_copy(emb_hbm.at[idx_v.at[1]], rows1, gs1).start()

    def _wb(rows, sem, j):
        return pltpu.make_async_copy(
            rows, out_hbm.at[pl.ds(base + j * _KC, _KC)], sem)

    def body(g, carry):
        j0 = 2 * g
        j1 = j0 + 1
        pltpu.make_async_copy(emb_hbm.at[idx_v.at[j0]], rows0, gs0).wait()
        _wb(rows0, ws0, j0).start()
        pltpu.make_async_copy(emb_hbm.at[idx_v.at[j1]], rows1, gs1).wait()
        _wb(rows1, ws1, j1).start()

        @pl.when(j0 + 2 < _NCHUNK)
        def _():
            _wb(rows0, ws0, j0).wait()
            pltpu.make_async_copy(emb_hbm.at[idx_v.at[j0 + 2]], rows0, gs0).start()
            _wb(rows1, ws1, j1).wait()
            pltpu.make_async_copy(emb_hbm.at[idx_v.at[j1 + 2]], rows1, gs1).start()

        return carry

    lax.fori_loop(0, _NCHUNK // 2, body, 0)
    _wb(rows0, ws0, _NCHUNK - 2).wait()
    _wb(rows1, ws1, _NCHUNK - 1).wait()


_gather = functools.partial(
    pl.kernel,
    out_type=jax.ShapeDtypeStruct((_N, _EMBED), jnp.float32),
    mesh=plsc.VectorSubcoreMesh(core_axis_name="c", subcore_axis_name="s"),
    scratch_types=[
        pltpu.VMEM((_NCHUNK, _KC), jnp.int32),
        pltpu.VMEM((_KC, _EMBED), jnp.float32),
        pltpu.VMEM((_KC, _EMBED), jnp.float32),
        pltpu.SemaphoreType.DMA,
        pltpu.SemaphoreType.DMA,
        pltpu.SemaphoreType.DMA,
        pltpu.SemaphoreType.DMA,
    ],
    compiler_params=pltpu.CompilerParams(use_tc_tiling_on_sc=False),
)(_gather_body)


_PBLK = 4096


def _linear_body(h_ref, w2_ref, b2_ref, o_ref):
    o_ref[...] = (
        jnp.dot(h_ref[...], w2_ref[...], preferred_element_type=jnp.float32)
        + b2_ref[...]
    )


def _linear(hp, w2, b2):
    np = _N // 2
    return pl.pallas_call(
        _linear_body,
        grid=(np // _PBLK,),
        in_specs=[
            pl.BlockSpec((_PBLK, 128), lambda i: (i, 0)),
            pl.BlockSpec((128, 128), lambda i: (0, 0)),
            pl.BlockSpec((1, 128), lambda i: (0, 0)),
        ],
        out_specs=pl.BlockSpec((_PBLK, 128), lambda i: (i, 0)),
        out_shape=jax.ShapeDtypeStruct((np, 128), jnp.float32),
    )(hp, w2, b2)


def kernel(x, emb, W, b):
    idx = x.reshape(-1).astype(jnp.int32).reshape(_NW, _NCHUNK, _KC)
    h = _gather(emb, idx)
    hp = h.reshape(_N // 2, 128)
    wt = W.T
    w2 = (
        jnp.zeros((128, 128), jnp.float32)
        .at[:_EMBED, :_EMBED].set(wt)
        .at[_EMBED:, _EMBED:].set(wt)
    )
    b2 = jnp.concatenate([b, b]).reshape(1, 128)
    op = _linear(hp, w2, b2)
    return op.reshape(_B, _L, _EMBED)


# speedup vs baseline: 2.0798x; 1.3185x over previous
import functools

import jax
import jax.numpy as jnp
from jax import lax
from jax.experimental import pallas as pl
from jax.experimental.pallas import tpu as pltpu
from jax.experimental.pallas import tpu_sc as plsc

_VOCAB = 1000000
_EMBED = 64
_B = 16384
_L = 50
_N = _B * _L

_NC = 2
_NS = 16
_NW = _NC * _NS
_KC = 128
_PER_W = _N // _NW
_NCHUNK = _PER_W // _KC


def _gather_body(emb_hbm, idx_hbm, out_hbm, idx_v, rows0, rows1,
                 gs0, gs1, ws0, ws1):
    wid = lax.axis_index("s") * _NC + lax.axis_index("c")
    base = wid * _PER_W
    pltpu.sync_copy(idx_hbm.at[wid], idx_v)

    pltpu.make_async_copy(emb_hbm.at[idx_v.at[0]], rows0, gs0).start()
    pltpu.make_async_copy(emb_hbm.at[idx_v.at[1]], rows1, gs1).start()

    def _wb(rows, sem, j):
        return pltpu.make_async_copy(
            rows, out_hbm.at[pl.ds(base + j * _KC, _KC)], sem)

    def body(g, carry):
        j0 = 2 * g
        j1 = j0 + 1
        pltpu.make_async_copy(emb_hbm.at[idx_v.at[j0]], rows0, gs0).wait()
        _wb(rows0, ws0, j0).start()
        pltpu.make_async_copy(emb_hbm.at[idx_v.at[j1]], rows1, gs1).wait()
        _wb(rows1, ws1, j1).start()

        @pl.when(j0 + 2 < _NCHUNK)
        def _():
            _wb(rows0, ws0, j0).wait()
            pltpu.make_async_copy(emb_hbm.at[idx_v.at[j0 + 2]], rows0, gs0).start()
            _wb(rows1, ws1, j1).wait()
            pltpu.make_async_copy(emb_hbm.at[idx_v.at[j1 + 2]], rows1, gs1).start()

        return carry

    lax.fori_loop(0, _NCHUNK // 2, body, 0)
    _wb(rows0, ws0, _NCHUNK - 2).wait()
    _wb(rows1, ws1, _NCHUNK - 1).wait()


_gather = functools.partial(
    pl.kernel,
    out_type=jax.ShapeDtypeStruct((_N, _EMBED), jnp.float32),
    mesh=plsc.VectorSubcoreMesh(core_axis_name="c", subcore_axis_name="s"),
    scratch_types=[
        pltpu.VMEM((_NCHUNK, _KC), jnp.int32),
        pltpu.VMEM((_KC, _EMBED), jnp.float32),
        pltpu.VMEM((_KC, _EMBED), jnp.float32),
        pltpu.SemaphoreType.DMA,
        pltpu.SemaphoreType.DMA,
        pltpu.SemaphoreType.DMA,
        pltpu.SemaphoreType.DMA,
    ],
    compiler_params=pltpu.CompilerParams(use_tc_tiling_on_sc=False),
)(_gather_body)


_HB = _B // 2


def _linear_body(h_ref, w_ref, b_ref, o_ref):
    h = h_ref[...]
    w = w_ref[...]
    bb = b_ref[...]
    nt = (((1,), (1,)), ((), ()))
    lo = lax.dot_general(w, h[:, :_EMBED], nt,
                         preferred_element_type=jnp.float32)
    hi = lax.dot_general(w, h[:, _EMBED:], nt,
                         preferred_element_type=jnp.float32)
    o_ref[0, :, :_HB] = lo + bb
    o_ref[0, :, _HB:] = hi + bb


def _linear(hp, w, b1):
    return pl.pallas_call(
        _linear_body,
        grid=(_L,),
        in_specs=[
            pl.BlockSpec((_HB, 128), lambda i: (i, 0)),
            pl.BlockSpec((_EMBED, _EMBED), lambda i: (0, 0)),
            pl.BlockSpec((_EMBED, 1), lambda i: (0, 0)),
        ],
        out_specs=pl.BlockSpec((1, _EMBED, _B), lambda i: (i, 0, 0)),
        out_shape=jax.ShapeDtypeStruct((_L, _EMBED, _B), jnp.float32),
    )(hp, w, b1)


def kernel(x, emb, W, b):
    xi = jnp.transpose(
        x.astype(jnp.int32).T.reshape(_L, 2, _HB), (0, 2, 1)
    ).reshape(-1)
    idx = xi.reshape(_NW, _NCHUNK, _KC)
    h = _gather(emb, idx)
    hp = h.reshape(_N // 2, 128)
    ot = _linear(hp, W, b.reshape(_EMBED, 1))
    return jnp.transpose(ot, (2, 0, 1))


# speedup vs baseline: 2.5652x; 1.2334x over previous
import functools

import jax
import jax.numpy as jnp
from jax import lax
from jax.experimental import pallas as pl
from jax.experimental.pallas import tpu as pltpu
from jax.experimental.pallas import tpu_sc as plsc

_VOCAB = 1000000
_EMBED = 64
_B = 16384
_L = 50
_N = _B * _L

_NC = 2
_NS = 16
_NW = _NC * _NS
_KC = 128
_PER_W = _N // _NW
_NCHUNK = _PER_W // _KC


_NPAIR = _NCHUNK // 2
_NBUF = 4
_PROWS_W = _PER_W // 2


def _gather_body(emb_hbm, idx_hbm, out_hbm, idx_v, bufs, gls, grs):
    wid = lax.axis_index("s") * _NC + lax.axis_index("c")
    pbase = wid * _PROWS_W
    pltpu.sync_copy(idx_hbm.at[wid], idx_v)

    def _glo(t, buf, sem):
        return pltpu.make_async_copy(
            emb_hbm.at[idx_v.at[2 * t]], buf.at[0], sem)

    def _ghi(t, buf, sem):
        return pltpu.make_async_copy(
            emb_hbm.at[idx_v.at[2 * t + 1]], buf.at[1], sem)

    def _wlo(buf, sem, t):
        return pltpu.make_async_copy(
            buf.at[0], out_hbm.at[pl.ds(pbase + t * _KC, _KC), 0:_EMBED], sem)

    def _whi(buf, sem, t):
        return pltpu.make_async_copy(
            buf.at[1], out_hbm.at[pl.ds(pbase + t * _KC, _KC), _EMBED:128], sem)

    for i in range(_NBUF):
        _glo(i, bufs[i], gls[i]).start()
        _ghi(i, bufs[i], grs[i]).start()

    def body(g, carry):
        t0 = _NBUF * g
        for i in range(_NBUF):
            t = t0 + i
            _glo(t, bufs[i], gls[i]).wait()
            _wlo(bufs[i], gls[i], t).start()
            _ghi(t, bufs[i], grs[i]).wait()
            _whi(bufs[i], grs[i], t).start()

        @pl.when(t0 + _NBUF < _NPAIR)
        def _():
            for i in range(_NBUF):
                t = t0 + i
                _wlo(bufs[i], gls[i], t).wait()
                _glo(t + _NBUF, bufs[i], gls[i]).start()
                _whi(bufs[i], grs[i], t).wait()
                _ghi(t + _NBUF, bufs[i], grs[i]).start()

        return carry

    lax.fori_loop(0, _NPAIR // _NBUF, body, 0)
    for i in range(_NBUF):
        t = _NPAIR - _NBUF + i
        _wlo(bufs[i], gls[i], t).wait()
        _whi(bufs[i], grs[i], t).wait()


def _gather_entry(emb_hbm, idx_hbm, out_hbm, idx_v, *sems):
    bufs = sems[:_NBUF]
    gls = sems[_NBUF:2 * _NBUF]
    grs = sems[2 * _NBUF:]
    _gather_body(emb_hbm, idx_hbm, out_hbm, idx_v, bufs, gls, grs)


_gather = functools.partial(
    pl.kernel,
    out_type=jax.ShapeDtypeStruct((_N // 2, 128), jnp.float32),
    mesh=plsc.VectorSubcoreMesh(core_axis_name="c", subcore_axis_name="s"),
    scratch_types=(
        [pltpu.VMEM((_NCHUNK, _KC), jnp.int32)]
        + [pltpu.VMEM((2, _KC, _EMBED), jnp.float32)] * _NBUF
        + [pltpu.SemaphoreType.DMA] * (2 * _NBUF)
    ),
    compiler_params=pltpu.CompilerParams(use_tc_tiling_on_sc=False),
)(_gather_entry)


_HB = _B // 2


def _linear_body(h_ref, w_ref, b_ref, o_ref):
    h = h_ref[...]
    w = w_ref[...]
    bb = b_ref[...]
    nt = (((1,), (1,)), ((), ()))
    lo = lax.dot_general(w, h[:, :_EMBED], nt,
                         preferred_element_type=jnp.float32)
    hi = lax.dot_general(w, h[:, _EMBED:], nt,
                         preferred_element_type=jnp.float32)
    o_ref[0, :, :_HB] = lo + bb
    o_ref[0, :, _HB:] = hi + bb


def _linear(hp, w, b1):
    return pl.pallas_call(
        _linear_body,
        grid=(_L,),
        in_specs=[
            pl.BlockSpec((_HB, 128), lambda i: (i, 0)),
            pl.BlockSpec((_EMBED, _EMBED), lambda i: (0, 0)),
            pl.BlockSpec((_EMBED, 1), lambda i: (0, 0)),
        ],
        out_specs=pl.BlockSpec((1, _EMBED, _B), lambda i: (i, 0, 0)),
        out_shape=jax.ShapeDtypeStruct((_L, _EMBED, _B), jnp.float32),
    )(hp, w, b1)


def kernel(x, emb, W, b):
    xi = jnp.transpose(
        x.astype(jnp.int32).T.reshape(_L, 2, _HB // _KC, _KC), (0, 2, 1, 3)
    ).reshape(-1)
    idx = xi.reshape(_NW, _NCHUNK, _KC)
    hp = _gather(emb, idx)
    ot = _linear(hp, W, b.reshape(_EMBED, 1))
    return jnp.transpose(ot, (2, 0, 1))
